# baseline (speedup 1.0000x reference)
import functools

import jax
import jax.numpy as jnp
from jax.experimental import pallas as pl
from jax.experimental.pallas import tpu as pltpu

_NSPLIT = 4
_VMEM_LIMIT = 8 * 1024 * 1024


def _se_body(w1t_ref, b1_ref, w2_ref, b2_ref, x_ref, o_ref, *, inv_s):
    xs = x_ref[...]
    pooled = jnp.sum(xs, axis=-1, dtype=jnp.float32) * inv_s
    hid = jnp.sum(w1t_ref[...][None, :, :] * pooled[:, :, None], axis=1) \
        + b1_ref[...]
    hid = jnp.maximum(hid, 0.0)
    z = jnp.sum(w2_ref[...][None, :, :] * hid[:, None, :], axis=-1) \
        + b2_ref[...][:, 0][None, :]
    gate = 1.0 / (1.0 + jnp.exp(-z))
    o_ref[...] = xs * gate.astype(xs.dtype)[:, :, None]


def kernel(x, w1, b1, w2, b2):
    b, c, d, hh, ww = x.shape
    s = d * hh * ww
    hidden = w1.shape[0]
    inv_s = 1.0 / float(s)

    x_flat = x.reshape(b, c, s)
    w1t = jnp.asarray(w1, jnp.float32).T
    b1r = jnp.asarray(b1, jnp.float32).reshape(1, hidden)
    w2m = jnp.asarray(w2, jnp.float32)
    b2c = jnp.asarray(b2, jnp.float32).reshape(c, 1)

    nsplit = _NSPLIT
    bb = (b + nsplit - 1) // nsplit

    call = pl.pallas_call(
        functools.partial(_se_body, inv_s=inv_s),
        out_shape=jax.ShapeDtypeStruct((bb, c, s), x.dtype),
        in_specs=[
            pl.BlockSpec(memory_space=pltpu.MemorySpace.VMEM),
            pl.BlockSpec(memory_space=pltpu.MemorySpace.VMEM),
            pl.BlockSpec(memory_space=pltpu.MemorySpace.VMEM),
            pl.BlockSpec(memory_space=pltpu.MemorySpace.VMEM),
            pl.BlockSpec(memory_space=pltpu.MemorySpace.VMEM),
        ],
        out_specs=pl.BlockSpec(memory_space=pltpu.MemorySpace.VMEM),
        compiler_params=pltpu.CompilerParams(
            vmem_limit_bytes=_VMEM_LIMIT),
        cost_estimate=pl.CostEstimate(
            flops=2 * bb * c * s, transcendentals=bb * c,
            bytes_accessed=2 * bb * c * s * 4),
    )

    parts = []
    for h in range(nsplit):
        lo = h * bb
        hi = min(lo + bb, b)
        xh = jax.lax.slice(x_flat, (lo, 0, 0), (hi, c, s))
        if hi - lo < bb:
            xh = jnp.pad(xh, ((0, bb - (hi - lo)), (0, 0), (0, 0)))
        yh = call(w1t, b1r, w2m, b2c, xh)
        parts.append(yh[:hi - lo])

    out_flat = jnp.concatenate(parts, axis=0)
    return out_flat.reshape(b, c, d, hh, ww)


# speedup vs baseline: 1.5390x; 1.5390x over previous
import functools

import jax
import jax.numpy as jnp
from jax.experimental import pallas as pl
from jax.experimental.pallas import tpu as pltpu

_NBUF = 8
_VMEM_LIMIT = 40 * 1024 * 1024


def _ring_body(w1t_ref, b1_ref, w2_ref, b2_ref, x_ref, o_ref,
               xbuf, obuf, in_sem, out_sem, *, n, inv_s):
    nbuf = _NBUF

    def start_in(chunk):
        pltpu.make_async_copy(
            x_ref.at[chunk], xbuf.at[chunk % nbuf],
            in_sem.at[chunk % nbuf]).start()

    def wait_in(slot):
        pltpu.make_async_copy(
            x_ref.at[0], xbuf.at[slot], in_sem.at[slot]).wait()

    def start_out(chunk):
        pltpu.make_async_copy(
            obuf.at[chunk % nbuf], o_ref.at[chunk],
            out_sem.at[chunk % nbuf]).start()

    def wait_out(slot):
        pltpu.make_async_copy(
            obuf.at[slot], o_ref.at[0], out_sem.at[slot]).wait()

    for k in range(min(nbuf - 1, n)):
        start_in(k)

    for i in range(n):
        slot = i % nbuf

        if i + nbuf - 1 < n:
            start_in(i + nbuf - 1)

        wait_in(slot)

        if i >= nbuf:
            wait_out(slot)

        xs = xbuf[slot]
        t = jnp.sum(xs, axis=1, dtype=jnp.float32)
        pooled = jnp.sum(t, axis=-1, keepdims=True) * inv_s
        hid = jnp.sum(w1t_ref[...] * pooled, axis=0,
                      keepdims=True) + b1_ref[...]
        hid = jnp.maximum(hid, 0.0)
        z = jnp.sum(w2_ref[...] * hid, axis=-1,
                    keepdims=True) + b2_ref[...]
        gate = 1.0 / (1.0 + jnp.exp(-z))
        obuf[slot] = xs * gate.astype(xs.dtype)[:, :, None]

        start_out(i)

    for k in range(max(n - nbuf, 0), n):
        wait_out(k % nbuf)


def kernel(x, w1, b1, w2, b2):
    b, c, d, hh, ww = x.shape
    s = d * hh * ww
    hidden = w1.shape[0]
    inv_s = 1.0 / float(s)
    r = s // 128

    x4 = x.reshape(b, c, r, 128)
    w1t = jnp.asarray(w1, jnp.float32).T
    b1r = jnp.asarray(b1, jnp.float32).reshape(1, hidden)
    w2m = jnp.asarray(w2, jnp.float32)
    b2c = jnp.asarray(b2, jnp.float32).reshape(c, 1)

    out4 = pl.pallas_call(
        functools.partial(_ring_body, n=b, inv_s=inv_s),
        out_shape=jax.ShapeDtypeStruct((b, c, r, 128), x.dtype),
        in_specs=[
            pl.BlockSpec(memory_space=pltpu.MemorySpace.VMEM),
            pl.BlockSpec(memory_space=pltpu.MemorySpace.VMEM),
            pl.BlockSpec(memory_space=pltpu.MemorySpace.VMEM),
            pl.BlockSpec(memory_space=pltpu.MemorySpace.VMEM),
            pl.BlockSpec(memory_space=pl.ANY),
        ],
        out_specs=pl.BlockSpec(memory_space=pl.ANY),
        scratch_shapes=[
            pltpu.VMEM((_NBUF, c, r, 128), x.dtype),
            pltpu.VMEM((_NBUF, c, r, 128), x.dtype),
            pltpu.SemaphoreType.DMA((_NBUF,)),
            pltpu.SemaphoreType.DMA((_NBUF,)),
        ],
        compiler_params=pltpu.CompilerParams(
            vmem_limit_bytes=_VMEM_LIMIT),
        cost_estimate=pl.CostEstimate(
            flops=2 * b * c * s, transcendentals=b * c,
            bytes_accessed=2 * b * c * s * 4),
    )(w1t, b1r, w2m, b2c, x4)

    return out4.reshape(b, c, d, hh, ww)


# speedup vs baseline: 1.5915x; 1.0341x over previous
import functools

import jax
import jax.numpy as jnp
from jax.experimental import pallas as pl
from jax.experimental.pallas import tpu as pltpu

_NBUF = 6
_NPRIO = 2
_VMEM_LIMIT = 40 * 1024 * 1024


def _ring_body(w1t_ref, b1_ref, w2_ref, b2_ref, x_ref, o_ref,
               xbuf, obuf, in_sem, out_sem, *, n, cb, inv_s):
    nbuf = _NBUF

    def start_in(chunk):
        pltpu.make_async_copy(
            x_ref.at[pl.ds(chunk * cb, cb)], xbuf.at[chunk % nbuf],
            in_sem.at[chunk % nbuf]).start(priority=chunk % _NPRIO)

    def wait_in(slot):
        pltpu.make_async_copy(
            x_ref.at[pl.ds(0, cb)], xbuf.at[slot], in_sem.at[slot]).wait()

    def start_out(chunk):
        pltpu.make_async_copy(
            obuf.at[chunk % nbuf], o_ref.at[pl.ds(chunk * cb, cb)],
            out_sem.at[chunk % nbuf]).start(priority=chunk % _NPRIO)

    def wait_out(slot):
        pltpu.make_async_copy(
            obuf.at[slot], o_ref.at[pl.ds(0, cb)], out_sem.at[slot]).wait()

    for k in range(min(nbuf - 1, n)):
        start_in(k)

    for i in range(n):
        slot = i % nbuf

        if i + nbuf - 1 < n:
            start_in(i + nbuf - 1)

        wait_in(slot)

        if i >= nbuf:
            wait_out(slot)

        xs = xbuf[slot]
        pooled = jnp.sum(xs, axis=-1, dtype=jnp.float32) * inv_s
        w1t = w1t_ref[...]
        hid = jnp.sum(w1t[None, :, :] * pooled[:, :, None], axis=1) \
            + b1_ref[...]
        hid = jnp.maximum(hid, 0.0)
        z = jnp.sum(w2_ref[...][None, :, :] * hid[:, None, :], axis=-1) \
            + b2_ref[...][:, 0][None, :]
        gate = 1.0 / (1.0 + jnp.exp(-z))
        obuf[slot] = xs * gate.astype(xs.dtype)[:, :, None]

        start_out(i)

    for k in range(max(n - nbuf, 0), n):
        wait_out(k % nbuf)


def kernel(x, w1, b1, w2, b2):
    b, c, d, hh, ww = x.shape
    s = d * hh * ww
    hidden = w1.shape[0]
    inv_s = 1.0 / float(s)

    x_flat = x.reshape(b, c, s)
    w1t = jnp.asarray(w1, jnp.float32).T
    b1r = jnp.asarray(b1, jnp.float32).reshape(1, hidden)
    w2m = jnp.asarray(w2, jnp.float32)
    b2c = jnp.asarray(b2, jnp.float32).reshape(c, 1)

    cb = 2
    n = b // cb

    out_flat = pl.pallas_call(
        functools.partial(_ring_body, n=n, cb=cb, inv_s=inv_s),
        out_shape=jax.ShapeDtypeStruct((b, c, s), x.dtype),
        in_specs=[
            pl.BlockSpec(memory_space=pltpu.MemorySpace.VMEM),
            pl.BlockSpec(memory_space=pltpu.MemorySpace.VMEM),
            pl.BlockSpec(memory_space=pltpu.MemorySpace.VMEM),
            pl.BlockSpec(memory_space=pltpu.MemorySpace.VMEM),
            pl.BlockSpec(memory_space=pl.ANY),
        ],
        out_specs=pl.BlockSpec(memory_space=pl.ANY),
        scratch_shapes=[
            pltpu.VMEM((_NBUF, cb, c, s), x.dtype),
            pltpu.VMEM((_NBUF, cb, c, s), x.dtype),
            pltpu.SemaphoreType.DMA((_NBUF,)),
            pltpu.SemaphoreType.DMA((_NBUF,)),
        ],
        compiler_params=pltpu.CompilerParams(
            vmem_limit_bytes=_VMEM_LIMIT),
        cost_estimate=pl.CostEstimate(
            flops=2 * b * c * s, transcendentals=b * c,
            bytes_accessed=2 * b * c * s * 4),
    )(w1t, b1r, w2m, b2c, x_flat)

    return out_flat.reshape(b, c, d, hh, ww)
